# baseline (speedup 1.0000x reference)
import functools

import jax
import jax.numpy as jnp
from jax import lax
from jax.experimental import pallas as pl
from jax.experimental.pallas import tpu as pltpu
from jax.experimental.pallas import tpu_sc as plsc

N = 50000
E = 800000
D = 64
H = 32
L = 1024
NC = 2
NS = 16
BATCH = 128
CHUNK = 20
_NB0 = -(-(E // NS) // BATCH)
NB = -(-_NB0 // CHUNK) * CHUNK
EPW = NB * BATCH
EPAD = NS * EPW - E
TRASH = 48
RPS = -(-(N + TRASH) // NS)
NACC = NS * RPS
RLAST = N - (NS - 1) * RPS


def _make_segsum(chunk=CHUNK):
    assert NB % chunk == 0 and chunk % 4 == 0
    nch = NB // chunk
    assert nch % 2 == 0
    mesh = plsc.VectorSubcoreMesh(
        core_axis_name="c", subcore_axis_name="s", num_cores=NC, num_subcores=NS
    )

    @functools.partial(
        pl.kernel,
        out_type=jax.ShapeDtypeStruct((NC, N, H), jnp.float32),
        mesh=mesh,
        scratch_types=[
            pltpu.VMEM((chunk, BATCH), jnp.int32),
            pltpu.VMEM((chunk, BATCH), jnp.int32),
            pltpu.VMEM((chunk, BATCH), jnp.int32),
            pltpu.VMEM((chunk, BATCH), jnp.int32),
            pltpu.VMEM((BATCH, H), jnp.float32),
            pltpu.VMEM((BATCH, H), jnp.float32),
            pltpu.VMEM((BATCH, H), jnp.float32),
            pltpu.VMEM((BATCH, H), jnp.float32),
            pltpu.VMEM_SHARED((NACC, H), jnp.float32),
            pltpu.SemaphoreType.DMA,
            pltpu.SemaphoreType.DMA,
            pltpu.SemaphoreType.DMA,
            pltpu.SemaphoreType.DMA,
            pltpu.SemaphoreType.DMA,
            pltpu.SemaphoreType.DMA,
            pltpu.SemaphoreType.DMA,
            pltpu.SemaphoreType.DMA,
            pltpu.SemaphoreType.DMA,
            pltpu.SemaphoreType.DMA,
        ],
        compiler_params=pltpu.CompilerParams(use_tc_tiling_on_sc=False),
    )
    def segsum(hf_hbm, src_hbm, dst_hbm, zeros_hbm, out_hbm,
               idx_s0, idx_d0, idx_s1, idx_d1, r0, r1, r2, r3, acc,
               gs0, gs1, gs2, gs3, ss0, ss1, ss2, ss3, is0, is1):
        c = lax.axis_index("c")
        s = lax.axis_index("s")
        w = c * NS + s
        rows = (r0, r1, r2, r3)
        gsem = (gs0, gs1, gs2, gs3)
        ssem = (ss0, ss1, ss2, ss3)
        pltpu.sync_copy(zeros_hbm, acc.at[pl.ds(s * RPS, RPS)])
        plsc.subcore_barrier()

        def stage(k, idxs, idxd, isem):
            pltpu.async_copy(src_hbm.at[w, pl.ds(k * chunk, chunk)], idxs,
                             isem)
            pltpu.async_copy(dst_hbm.at[w, pl.ds(k * chunk, chunk)], idxd,
                             isem)

        def run_chunk(k, idxs, idxd, isem, pre):
            if pre is not None:
                cond, pk, pidxs, pidxd, pisem = pre
                pl.when(cond)(lambda: stage(pk, pidxs, pidxd, pisem))
            pltpu.make_async_copy(src_hbm.at[w, pl.ds(k * chunk, chunk)],
                                  idxs, isem).wait()
            pltpu.make_async_copy(dst_hbm.at[w, pl.ds(k * chunk, chunk)],
                                  idxd, isem).wait()
            pltpu.async_copy(hf_hbm.at[idxs.at[0]], rows[0], gsem[0])
            pltpu.async_copy(hf_hbm.at[idxs.at[1]], rows[1], gsem[1])

            def quad(q, cc):
                for i in range(4):
                    j = 4 * q + i
                    b = i
                    b2 = (i + 2) % 4
                    pltpu.make_async_copy(hf_hbm.at[idxs.at[j]], rows[b],
                                          gsem[b]).wait()
                    pltpu.async_copy(rows[b], acc.at[idxd.at[j]], ssem[b],
                                     add=True)
                    def _wait_prev(b2=b2, j=j):
                        pltpu.make_async_copy(rows[b2], acc.at[idxd.at[j - 2]],
                                              ssem[b2]).wait()

                    def _next_gather(b2=b2, j=j):
                        pltpu.async_copy(hf_hbm.at[idxs.at[j + 2]], rows[b2],
                                         gsem[b2])

                    pl.when(j >= 2)(_wait_prev)
                    pl.when(j < chunk - 2)(_next_gather)
                return cc

            lax.fori_loop(0, chunk // 4, quad, 0, unroll=False)
            pltpu.make_async_copy(rows[2], acc.at[idxd.at[chunk - 2]],
                                  ssem[2]).wait()
            pltpu.make_async_copy(rows[3], acc.at[idxd.at[chunk - 1]],
                                  ssem[3]).wait()

        stage(0, idx_s0, idx_d0, is0)

        def pairloop(k2, cc):
            k0 = 2 * k2
            run_chunk(k0, idx_s0, idx_d0, is0,
                      (k0 + 1 < nch, k0 + 1, idx_s1, idx_d1, is1))
            run_chunk(k0 + 1, idx_s1, idx_d1, is1,
                      (k2 < nch // 2 - 1, k0 + 2, idx_s0, idx_d0, is0))
            return cc

        lax.fori_loop(0, nch // 2, pairloop, 0, unroll=False)
        plsc.subcore_barrier()

        @pl.when(s < NS - 1)
        def _():
            pltpu.sync_copy(acc.at[pl.ds(s * RPS, RPS)],
                            out_hbm.at[c, pl.ds(s * RPS, RPS)])

        @pl.when(s == NS - 1)
        def _():
            pltpu.sync_copy(acc.at[pl.ds((NS - 1) * RPS, RLAST)],
                            out_hbm.at[c, pl.ds((NS - 1) * RPS, RLAST)])

    return segsum


_segsum = _make_segsum()


def _layer_body(ope_ref, h_ref, agg_ref, w_ref, b_ref, wr_ref, br_ref,
                oa_ref, ob_ref):
    h = h_ref[...]
    agg = jnp.concatenate([agg_ref[0], agg_ref[1]], axis=-1)
    g = jnp.dot(ope_ref[0, 0] * h + agg, w_ref[...],
                preferred_element_type=jnp.float32) + b_ref[...]
    r = jnp.maximum(g, 0.0)
    t = jnp.dot(r, wr_ref[...], preferred_element_type=jnp.float32) + br_ref[...]
    hn = jnp.maximum(t, 0.0) + r
    oa_ref[...] = hn[:, :H]
    ob_ref[...] = hn[:, H:]


def _proj_body(ha_ref, hb_ref, wout_ref, bout_ref, p_ref):
    h = jnp.concatenate([ha_ref[...], hb_ref[...]], axis=-1)
    p_ref[...] = jnp.dot(0.5 * h, wout_ref[...],
                         preferred_element_type=jnp.float32) + bout_ref[...]


def _final_body(ope_ref, ha_ref, hb_ref, agg_ref, p_ref, w_ref, b_ref,
                wr_ref, br_ref, wout_ref, out_ref):
    h = jnp.concatenate([ha_ref[...], hb_ref[...]], axis=-1)
    agg = jnp.concatenate([agg_ref[0], agg_ref[1]], axis=-1)
    g = jnp.dot(ope_ref[0, 0] * h + agg, w_ref[...],
                preferred_element_type=jnp.float32) + b_ref[...]
    r = jnp.maximum(g, 0.0)
    t = jnp.dot(r, wr_ref[...], preferred_element_type=jnp.float32) + br_ref[...]
    h2 = jnp.maximum(t, 0.0) + r
    out_ref[...] = p_ref[...] + jnp.dot(
        0.5 * h2, wout_ref[...], preferred_element_type=jnp.float32)


_R1 = 2000
_R2 = 1000

_smem_spec = pl.BlockSpec(memory_space=pltpu.SMEM)


def _const_spec(shape):
    return pl.BlockSpec(shape, lambda i: (0,) * len(shape))


_layer_call = pl.pallas_call(
    _layer_body,
    grid=(N // _R1,),
    in_specs=[
        _smem_spec,
        pl.BlockSpec((_R1, D), lambda i: (i, 0)),
        pl.BlockSpec((NC, _R1, H), lambda i: (0, i, 0)),
        _const_spec((D, D)),
        _const_spec((1, D)),
        _const_spec((D, D)),
        _const_spec((1, D)),
    ],
    out_specs=[
        pl.BlockSpec((_R1, H), lambda i: (i, 0)),
        pl.BlockSpec((_R1, H), lambda i: (i, 0)),
    ],
    out_shape=[
        jax.ShapeDtypeStruct((N, H), jnp.float32),
        jax.ShapeDtypeStruct((N, H), jnp.float32),
    ],
)

_proj_call = pl.pallas_call(
    _proj_body,
    grid=(N // _R2,),
    in_specs=[
        pl.BlockSpec((_R2, H), lambda i: (i, 0)),
        pl.BlockSpec((_R2, H), lambda i: (i, 0)),
        _const_spec((D, L)),
        _const_spec((1, L)),
    ],
    out_specs=pl.BlockSpec((_R2, L), lambda i: (i, 0)),
    out_shape=jax.ShapeDtypeStruct((N, L), jnp.float32),
)

_final_call = pl.pallas_call(
    _final_body,
    grid=(N // _R2,),
    in_specs=[
        _smem_spec,
        pl.BlockSpec((_R2, H), lambda i: (i, 0)),
        pl.BlockSpec((_R2, H), lambda i: (i, 0)),
        pl.BlockSpec((NC, _R2, H), lambda i: (0, i, 0)),
        pl.BlockSpec((_R2, L), lambda i: (i, 0)),
        _const_spec((D, D)),
        _const_spec((1, D)),
        _const_spec((D, D)),
        _const_spec((1, D)),
        _const_spec((D, L)),
    ],
    out_specs=pl.BlockSpec((_R2, L), lambda i: (i, 0)),
    out_shape=jax.ShapeDtypeStruct((N, L), jnp.float32),
)


def kernel(inputs, edge_index, eps1, W1, b1, Wr1, br1, eps2, W2, b2, Wr2, br2,
           Wout, bout):
    src = edge_index[0]
    dst = edge_index[1]
    pad = jnp.arange(EPAD, dtype=jnp.int32)
    src3 = jnp.concatenate([src, pad % N]).reshape(NS, NB, BATCH)
    src4 = jnp.concatenate([src3, src3 + N])
    dst3 = jnp.concatenate([dst, N + pad % TRASH]).reshape(NS, NB, BATCH)
    dst4 = jnp.concatenate([dst3, dst3])
    zeros = jnp.zeros((RPS, H), jnp.float32)

    h0f = jnp.concatenate([inputs[:, :H], inputs[:, H:]], axis=0)
    agg1 = _segsum(h0f, src4, dst4, zeros)

    ope1 = (1.0 + eps1).reshape(1, 1)
    ope2 = (1.0 + eps2).reshape(1, 1)
    h1a, h1b = _layer_call(ope1, inputs, agg1,
                           W1, b1.reshape(1, D), Wr1, br1.reshape(1, D))

    h1f = jnp.concatenate([h1a, h1b], axis=0)
    agg2 = _segsum(h1f, src4, dst4, zeros)
    p1 = _proj_call(h1a, h1b, Wout, bout.reshape(1, L))

    out = _final_call(ope2, h1a, h1b, agg2, p1,
                      W2, b2.reshape(1, D), Wr2, br2.reshape(1, D),
                      Wout)
    return out


# speedup vs baseline: 1.2346x; 1.2346x over previous
import functools

import jax
import jax.numpy as jnp
from jax import lax
from jax.experimental import pallas as pl
from jax.experimental.pallas import tpu as pltpu
from jax.experimental.pallas import tpu_sc as plsc

N = 50000
E = 800000
D = 64
H = 32
L = 1024
NC = 2
NS = 16
BATCH = 128
CHUNK = 20
_NB0 = -(-(E // NS) // BATCH)
NB = -(-_NB0 // CHUNK) * CHUNK
EPW = NB * BATCH
EPAD = NS * EPW - E
TRASH = 48
RPS = -(-(N + TRASH) // NS)
NACC = NS * RPS
RLAST = N - (NS - 1) * RPS


def _make_segsum(chunk=CHUNK):
    assert NB % chunk == 0 and chunk % 5 == 0
    nch = NB // chunk
    assert nch % 2 == 0
    mesh = plsc.VectorSubcoreMesh(
        core_axis_name="c", subcore_axis_name="s", num_cores=NC, num_subcores=NS
    )

    @functools.partial(
        pl.kernel,
        out_type=jax.ShapeDtypeStruct((NC, N, H), jnp.float32),
        mesh=mesh,
        scratch_types=[
            pltpu.VMEM((chunk, BATCH), jnp.int32),
            pltpu.VMEM((chunk, BATCH), jnp.int32),
            pltpu.VMEM((chunk, BATCH), jnp.int32),
            pltpu.VMEM((chunk, BATCH), jnp.int32),
            pltpu.VMEM((BATCH, H), jnp.float32),
            pltpu.VMEM((BATCH, H), jnp.float32),
            pltpu.VMEM((BATCH, H), jnp.float32),
            pltpu.VMEM((BATCH, H), jnp.float32),
            pltpu.VMEM((BATCH, H), jnp.float32),
            pltpu.VMEM_SHARED((NACC, H), jnp.float32),
            pltpu.SemaphoreType.DMA,
            pltpu.SemaphoreType.DMA,
            pltpu.SemaphoreType.DMA,
            pltpu.SemaphoreType.DMA,
            pltpu.SemaphoreType.DMA,
            pltpu.SemaphoreType.DMA,
            pltpu.SemaphoreType.DMA,
            pltpu.SemaphoreType.DMA,
            pltpu.SemaphoreType.DMA,
            pltpu.SemaphoreType.DMA,
            pltpu.SemaphoreType.DMA,
            pltpu.SemaphoreType.DMA,
        ],
        compiler_params=pltpu.CompilerParams(use_tc_tiling_on_sc=False),
    )
    def segsum(hf_hbm, src_hbm, dst_hbm, zeros_hbm, out_hbm,
               idx_s0, idx_d0, idx_s1, idx_d1, r0, r1, r2, r3, r4, acc,
               gs0, gs1, gs2, gs3, gs4, ss0, ss1, ss2, ss3, ss4, is0, is1):
        c = lax.axis_index("c")
        s = lax.axis_index("s")
        w = c * NS + s
        rows = (r0, r1, r2, r3, r4)
        gsem = (gs0, gs1, gs2, gs3, gs4)
        ssem = (ss0, ss1, ss2, ss3, ss4)
        pltpu.sync_copy(zeros_hbm, acc.at[pl.ds(s * RPS, RPS)])
        plsc.subcore_barrier()

        def stage(k, idxs, idxd, isem):
            pltpu.async_copy(src_hbm.at[w, pl.ds(k * chunk, chunk)], idxs,
                             isem)
            pltpu.async_copy(dst_hbm.at[w, pl.ds(k * chunk, chunk)], idxd,
                             isem)

        def run_chunk(k, idxs, idxd, isem, pre):
            if pre is not None:
                cond, pk, pidxs, pidxd, pisem = pre
                pl.when(cond)(lambda: stage(pk, pidxs, pidxd, pisem))
            pltpu.make_async_copy(src_hbm.at[w, pl.ds(k * chunk, chunk)],
                                  idxs, isem).wait()
            pltpu.make_async_copy(dst_hbm.at[w, pl.ds(k * chunk, chunk)],
                                  idxd, isem).wait()
            pltpu.async_copy(hf_hbm.at[idxs.at[0]], rows[0], gsem[0])
            pltpu.async_copy(hf_hbm.at[idxs.at[1]], rows[1], gsem[1])
            pltpu.async_copy(hf_hbm.at[idxs.at[2]], rows[2], gsem[2])

            def quint(q, cc):
                for i in range(5):
                    j = 5 * q + i
                    b = i
                    b3 = (i + 3) % 5
                    pltpu.make_async_copy(hf_hbm.at[idxs.at[j]], rows[b],
                                          gsem[b]).wait()
                    pltpu.async_copy(rows[b], acc.at[idxd.at[j]], ssem[b],
                                     add=True)

                    def _wait_prev(b3=b3, j=j):
                        pltpu.make_async_copy(rows[b3], acc.at[idxd.at[j - 2]],
                                              ssem[b3]).wait()

                    def _next_gather(b3=b3, j=j):
                        pltpu.async_copy(hf_hbm.at[idxs.at[j + 3]], rows[b3],
                                         gsem[b3])

                    pl.when(j >= 2)(_wait_prev)
                    pl.when(j < chunk - 3)(_next_gather)
                return cc

            lax.fori_loop(0, chunk // 5, quint, 0, unroll=False)
            pltpu.make_async_copy(rows[(chunk - 2) % 5],
                                  acc.at[idxd.at[chunk - 2]],
                                  ssem[(chunk - 2) % 5]).wait()
            pltpu.make_async_copy(rows[(chunk - 1) % 5],
                                  acc.at[idxd.at[chunk - 1]],
                                  ssem[(chunk - 1) % 5]).wait()

        stage(0, idx_s0, idx_d0, is0)

        def pairloop(k2, cc):
            k0 = 2 * k2
            run_chunk(k0, idx_s0, idx_d0, is0,
                      (k0 + 1 < nch, k0 + 1, idx_s1, idx_d1, is1))
            run_chunk(k0 + 1, idx_s1, idx_d1, is1,
                      (k2 < nch // 2 - 1, k0 + 2, idx_s0, idx_d0, is0))
            return cc

        lax.fori_loop(0, nch // 2, pairloop, 0, unroll=False)
        plsc.subcore_barrier()

        @pl.when(s < NS - 1)
        def _():
            pltpu.sync_copy(acc.at[pl.ds(s * RPS, RPS)],
                            out_hbm.at[c, pl.ds(s * RPS, RPS)])

        @pl.when(s == NS - 1)
        def _():
            pltpu.sync_copy(acc.at[pl.ds((NS - 1) * RPS, RLAST)],
                            out_hbm.at[c, pl.ds((NS - 1) * RPS, RLAST)])

    return segsum


_segsum = _make_segsum()


def _layer_body(ope_ref, h_ref, agg_ref, w_ref, b_ref, wr_ref, br_ref,
                oa_ref, ob_ref):
    h = h_ref[...]
    agg = jnp.concatenate([agg_ref[0], agg_ref[1]], axis=-1)
    g = jnp.dot(ope_ref[0, 0] * h + agg, w_ref[...],
                preferred_element_type=jnp.float32) + b_ref[...]
    r = jnp.maximum(g, 0.0)
    t = jnp.dot(r, wr_ref[...], preferred_element_type=jnp.float32) + br_ref[...]
    hn = jnp.maximum(t, 0.0) + r
    oa_ref[...] = hn[:, :H]
    ob_ref[...] = hn[:, H:]


def _final_body(ope_ref, ha_ref, hb_ref, agg_ref, w_ref, b_ref,
                wr_ref, br_ref, wout_ref, bout_ref, out_ref):
    h = jnp.concatenate([ha_ref[...], hb_ref[...]], axis=-1)
    agg = jnp.concatenate([agg_ref[0], agg_ref[1]], axis=-1)
    g = jnp.dot(ope_ref[0, 0] * h + agg, w_ref[...],
                preferred_element_type=jnp.float32) + b_ref[...]
    r = jnp.maximum(g, 0.0)
    t = jnp.dot(r, wr_ref[...], preferred_element_type=jnp.float32) + br_ref[...]
    h2 = jnp.maximum(t, 0.0) + r
    m = (h + h2) * 0.5
    out_ref[...] = jnp.dot(m, wout_ref[...],
                           preferred_element_type=jnp.float32) + bout_ref[...]


_R1 = 2000
_R2 = 1000

_smem_spec = pl.BlockSpec(memory_space=pltpu.SMEM)


def _const_spec(shape):
    return pl.BlockSpec(shape, lambda i: (0,) * len(shape))


_layer_call = pl.pallas_call(
    _layer_body,
    grid=(N // _R1,),
    in_specs=[
        _smem_spec,
        pl.BlockSpec((_R1, D), lambda i: (i, 0)),
        pl.BlockSpec((NC, _R1, H), lambda i: (0, i, 0)),
        _const_spec((D, D)),
        _const_spec((1, D)),
        _const_spec((D, D)),
        _const_spec((1, D)),
    ],
    out_specs=[
        pl.BlockSpec((_R1, H), lambda i: (i, 0)),
        pl.BlockSpec((_R1, H), lambda i: (i, 0)),
    ],
    out_shape=[
        jax.ShapeDtypeStruct((N, H), jnp.float32),
        jax.ShapeDtypeStruct((N, H), jnp.float32),
    ],
)

_final_call = pl.pallas_call(
    _final_body,
    grid=(N // _R2,),
    in_specs=[
        _smem_spec,
        pl.BlockSpec((_R2, H), lambda i: (i, 0)),
        pl.BlockSpec((_R2, H), lambda i: (i, 0)),
        pl.BlockSpec((NC, _R2, H), lambda i: (0, i, 0)),
        _const_spec((D, D)),
        _const_spec((1, D)),
        _const_spec((D, D)),
        _const_spec((1, D)),
        _const_spec((D, L)),
        _const_spec((1, L)),
    ],
    out_specs=pl.BlockSpec((_R2, L), lambda i: (i, 0)),
    out_shape=jax.ShapeDtypeStruct((N, L), jnp.float32),
)


def kernel(inputs, edge_index, eps1, W1, b1, Wr1, br1, eps2, W2, b2, Wr2, br2,
           Wout, bout):
    src = edge_index[0]
    dst = edge_index[1]
    pad = jnp.arange(EPAD, dtype=jnp.int32)
    src3 = jnp.concatenate([src, pad % N]).reshape(NS, NB, BATCH)
    src4 = jnp.concatenate([src3, src3 + N])
    dst3 = jnp.concatenate([dst, N + pad % TRASH]).reshape(NS, NB, BATCH)
    dst4 = jnp.concatenate([dst3, dst3])
    zeros = jnp.zeros((RPS, H), jnp.float32)

    h0f = jnp.concatenate([inputs[:, :H], inputs[:, H:]], axis=0)
    agg1 = _segsum(h0f, src4, dst4, zeros)

    ope1 = (1.0 + eps1).reshape(1, 1)
    ope2 = (1.0 + eps2).reshape(1, 1)
    h1a, h1b = _layer_call(ope1, inputs, agg1,
                           W1, b1.reshape(1, D), Wr1, br1.reshape(1, D))

    h1f = jnp.concatenate([h1a, h1b], axis=0)
    agg2 = _segsum(h1f, src4, dst4, zeros)

    out = _final_call(ope2, h1a, h1b, agg2,
                      W2, b2.reshape(1, D), Wr2, br2.reshape(1, D),
                      Wout, bout.reshape(1, L))
    return out


# speedup vs baseline: 1.2507x; 1.0130x over previous
import functools

import jax
import jax.numpy as jnp
from jax import lax
from jax.experimental import pallas as pl
from jax.experimental.pallas import tpu as pltpu
from jax.experimental.pallas import tpu_sc as plsc

N = 50000
E = 800000
D = 64
H = 32
L = 1024
NC = 2
NS = 16
BATCH = 128
CHUNK = 20
_NB0 = -(-(E // NS) // BATCH)
NB = -(-_NB0 // CHUNK) * CHUNK
EPW = NB * BATCH
EPAD = NS * EPW - E
TRASH = 48
RPS = -(-(N + TRASH) // NS)
NACC = NS * RPS
RLAST = N - (NS - 1) * RPS


def _make_segsum(chunk=CHUNK):
    assert NB % chunk == 0 and chunk % 5 == 0
    nch = NB // chunk
    assert nch % 2 == 0
    mesh = plsc.VectorSubcoreMesh(
        core_axis_name="c", subcore_axis_name="s", num_cores=NC, num_subcores=NS
    )

    @functools.partial(
        pl.kernel,
        out_type=jax.ShapeDtypeStruct((NC, N, H), jnp.float32),
        mesh=mesh,
        scratch_types=[
            pltpu.VMEM((chunk, BATCH), jnp.int32),
            pltpu.VMEM((chunk, BATCH), jnp.int32),
            pltpu.VMEM((chunk, BATCH), jnp.int32),
            pltpu.VMEM((chunk, BATCH), jnp.int32),
            pltpu.VMEM((BATCH, H), jnp.float32),
            pltpu.VMEM((BATCH, H), jnp.float32),
            pltpu.VMEM((BATCH, H), jnp.float32),
            pltpu.VMEM((BATCH, H), jnp.float32),
            pltpu.VMEM((BATCH, H), jnp.float32),
            pltpu.VMEM_SHARED((NACC, H), jnp.float32),
            pltpu.SemaphoreType.DMA,
            pltpu.SemaphoreType.DMA,
            pltpu.SemaphoreType.DMA,
            pltpu.SemaphoreType.DMA,
            pltpu.SemaphoreType.DMA,
            pltpu.SemaphoreType.DMA,
            pltpu.SemaphoreType.DMA,
            pltpu.SemaphoreType.DMA,
            pltpu.SemaphoreType.DMA,
            pltpu.SemaphoreType.DMA,
            pltpu.SemaphoreType.DMA,
            pltpu.SemaphoreType.DMA,
        ],
        compiler_params=pltpu.CompilerParams(use_tc_tiling_on_sc=False),
    )
    def segsum(hf_hbm, src_hbm, dst_hbm, zeros_hbm, out_hbm,
               idx_s0, idx_d0, idx_s1, idx_d1, r0, r1, r2, r3, r4, acc,
               gs0, gs1, gs2, gs3, gs4, ss0, ss1, ss2, ss3, ss4, is0, is1):
        c = lax.axis_index("c")
        s = lax.axis_index("s")
        w = c * NS + s
        rows = (r0, r1, r2, r3, r4)
        gsem = (gs0, gs1, gs2, gs3, gs4)
        ssem = (ss0, ss1, ss2, ss3, ss4)
        pltpu.sync_copy(zeros_hbm, acc.at[pl.ds(s * RPS, RPS)])
        plsc.subcore_barrier()

        def stage(k, idxs, idxd, isem):
            pltpu.async_copy(src_hbm.at[w, pl.ds(k * chunk, chunk)], idxs,
                             isem)
            pltpu.async_copy(dst_hbm.at[w, pl.ds(k * chunk, chunk)], idxd,
                             isem)

        def run_chunk(k, idxs, idxd, isem, pre):
            if pre is not None:
                cond, pk, pidxs, pidxd, pisem = pre
                pl.when(cond)(lambda: stage(pk, pidxs, pidxd, pisem))
            pltpu.make_async_copy(src_hbm.at[w, pl.ds(k * chunk, chunk)],
                                  idxs, isem).wait()
            pltpu.make_async_copy(dst_hbm.at[w, pl.ds(k * chunk, chunk)],
                                  idxd, isem).wait()
            pltpu.async_copy(hf_hbm.at[idxs.at[0]], rows[0], gsem[0])
            pltpu.async_copy(hf_hbm.at[idxs.at[1]], rows[1], gsem[1])
            pltpu.async_copy(hf_hbm.at[idxs.at[2]], rows[2], gsem[2])

            def quint(q, cc):
                for i in range(5):
                    j = 5 * q + i
                    b = i
                    b3 = (i + 3) % 5
                    pltpu.make_async_copy(hf_hbm.at[idxs.at[j]], rows[b],
                                          gsem[b]).wait()
                    pltpu.async_copy(rows[b], acc.at[idxd.at[j]], ssem[b],
                                     add=True)

                    def _wait_prev(b3=b3, j=j):
                        pltpu.make_async_copy(rows[b3], acc.at[idxd.at[j - 2]],
                                              ssem[b3]).wait()

                    def _next_gather(b3=b3, j=j):
                        pltpu.async_copy(hf_hbm.at[idxs.at[j + 3]], rows[b3],
                                         gsem[b3])

                    pl.when(j >= 2)(_wait_prev)
                    pl.when(j < chunk - 3)(_next_gather)
                return cc

            lax.fori_loop(0, chunk // 5, quint, 0, unroll=False)
            pltpu.make_async_copy(rows[(chunk - 2) % 5],
                                  acc.at[idxd.at[chunk - 2]],
                                  ssem[(chunk - 2) % 5]).wait()
            pltpu.make_async_copy(rows[(chunk - 1) % 5],
                                  acc.at[idxd.at[chunk - 1]],
                                  ssem[(chunk - 1) % 5]).wait()

        stage(0, idx_s0, idx_d0, is0)

        def pairloop(k2, cc):
            k0 = 2 * k2
            run_chunk(k0, idx_s0, idx_d0, is0,
                      (k0 + 1 < nch, k0 + 1, idx_s1, idx_d1, is1))
            run_chunk(k0 + 1, idx_s1, idx_d1, is1,
                      (k2 < nch // 2 - 1, k0 + 2, idx_s0, idx_d0, is0))
            return cc

        lax.fori_loop(0, nch // 2, pairloop, 0, unroll=False)
        plsc.subcore_barrier()

        @pl.when(s < NS - 1)
        def _():
            pltpu.sync_copy(acc.at[pl.ds(s * RPS, RPS)],
                            out_hbm.at[c, pl.ds(s * RPS, RPS)])

        @pl.when(s == NS - 1)
        def _():
            pltpu.sync_copy(acc.at[pl.ds((NS - 1) * RPS, RLAST)],
                            out_hbm.at[c, pl.ds((NS - 1) * RPS, RLAST)])

    return segsum


_segsum = _make_segsum()


def _layer_body(ope_ref, h_ref, agg_ref, w_ref, b_ref, wr_ref, br_ref,
                oa_ref, ob_ref):
    h = h_ref[...]
    agg = jnp.concatenate([agg_ref[0], agg_ref[1]], axis=-1)
    g = jnp.dot(ope_ref[0, 0] * h + agg, w_ref[...],
                preferred_element_type=jnp.float32) + b_ref[...]
    r = jnp.maximum(g, 0.0)
    t = jnp.dot(r, wr_ref[...], preferred_element_type=jnp.float32) + br_ref[...]
    hn = jnp.maximum(t, 0.0) + r
    oa_ref[...] = hn[:, :H]
    ob_ref[...] = hn[:, H:]


def _final_body(ope_ref, ha_ref, hb_ref, agg_ref, w_ref, b_ref,
                wr_ref, br_ref, wout_ref, bout_ref, out_ref):
    h = jnp.concatenate([ha_ref[...], hb_ref[...]], axis=-1)
    agg = jnp.concatenate([agg_ref[0], agg_ref[1]], axis=-1)
    g = jnp.dot(ope_ref[0, 0] * h + agg, w_ref[...],
                preferred_element_type=jnp.float32) + b_ref[...]
    r = jnp.maximum(g, 0.0)
    t = jnp.dot(r, wr_ref[...], preferred_element_type=jnp.float32) + br_ref[...]
    h2 = jnp.maximum(t, 0.0) + r
    m = (h + h2) * 0.5
    out_ref[...] = jnp.dot(m, wout_ref[...],
                           preferred_element_type=jnp.float32) + bout_ref[...]


_R1 = 2000
_R2 = 2000

_smem_spec = pl.BlockSpec(memory_space=pltpu.SMEM)


def _const_spec(shape):
    return pl.BlockSpec(shape, lambda i: (0,) * len(shape))


_layer_call = pl.pallas_call(
    _layer_body,
    grid=(N // _R1,),
    in_specs=[
        _smem_spec,
        pl.BlockSpec((_R1, D), lambda i: (i, 0)),
        pl.BlockSpec((NC, _R1, H), lambda i: (0, i, 0)),
        _const_spec((D, D)),
        _const_spec((1, D)),
        _const_spec((D, D)),
        _const_spec((1, D)),
    ],
    out_specs=[
        pl.BlockSpec((_R1, H), lambda i: (i, 0)),
        pl.BlockSpec((_R1, H), lambda i: (i, 0)),
    ],
    out_shape=[
        jax.ShapeDtypeStruct((N, H), jnp.float32),
        jax.ShapeDtypeStruct((N, H), jnp.float32),
    ],
)

_final_call = pl.pallas_call(
    _final_body,
    grid=(N // _R2,),
    in_specs=[
        _smem_spec,
        pl.BlockSpec((_R2, H), lambda i: (i, 0)),
        pl.BlockSpec((_R2, H), lambda i: (i, 0)),
        pl.BlockSpec((NC, _R2, H), lambda i: (0, i, 0)),
        _const_spec((D, D)),
        _const_spec((1, D)),
        _const_spec((D, D)),
        _const_spec((1, D)),
        _const_spec((D, L)),
        _const_spec((1, L)),
    ],
    out_specs=pl.BlockSpec((_R2, L), lambda i: (i, 0)),
    out_shape=jax.ShapeDtypeStruct((N, L), jnp.float32),
)


def kernel(inputs, edge_index, eps1, W1, b1, Wr1, br1, eps2, W2, b2, Wr2, br2,
           Wout, bout):
    src = edge_index[0]
    dst = edge_index[1]
    pad = jnp.arange(EPAD, dtype=jnp.int32)
    src3 = jnp.concatenate([src, pad % N]).reshape(NS, NB, BATCH)
    src4 = jnp.concatenate([src3, src3 + N])
    dst3 = jnp.concatenate([dst, N + pad % TRASH]).reshape(NS, NB, BATCH)
    dst4 = jnp.concatenate([dst3, dst3])
    zeros = jnp.zeros((RPS, H), jnp.float32)

    h0f = jnp.concatenate([inputs[:, :H], inputs[:, H:]], axis=0)
    agg1 = _segsum(h0f, src4, dst4, zeros)

    ope1 = (1.0 + eps1).reshape(1, 1)
    ope2 = (1.0 + eps2).reshape(1, 1)
    h1a, h1b = _layer_call(ope1, inputs, agg1,
                           W1, b1.reshape(1, D), Wr1, br1.reshape(1, D))

    h1f = jnp.concatenate([h1a, h1b], axis=0)
    agg2 = _segsum(h1f, src4, dst4, zeros)

    out = _final_call(ope2, h1a, h1b, agg2,
                      W2, b2.reshape(1, D), Wr2, br2.reshape(1, D),
                      Wout, bout.reshape(1, L))
    return out
